# baseline (speedup 1.0000x reference)
import jax
import jax.numpy as jnp
from jax import lax
from jax.experimental import pallas as pl
from jax.experimental.pallas import tpu as pltpu
from jax.experimental.pallas import tpu_sc as plsc

C = 64
NLAT_IN, NLON_IN = 361, 720
NLAT_OUT, NLON_OUT = 721, 1440
L = 16
NS = 23
SPAD = NS * L
PAD = 2 * SPAD
NT = PAD // L
KB = 48
NBLK = NLON_IN // KB
OROWS = 16
NG = 2 * KB // OROWS
NW = 32
CPW = C // NW


def _body(x_hbm, wle_hbm, wlo_hbm, we_hbm, wo_hbm, out_hbm,
          xb0, xb1, lbuf, ob0, ob1, wlev, wlov, webv, wobv,
          sem0, sem1, semx0, semx1):
    cid = lax.axis_index("c")
    sid = lax.axis_index("s")
    wid = sid * 2 + cid

    pltpu.sync_copy(wle_hbm, wlev)
    pltpu.sync_copy(wlo_hbm, wlov)

    iot = lax.iota(jnp.int32, L)
    iot1 = iot + 1
    iot2 = 2 * iot
    last = jnp.full((L,), NLAT_OUT - 1, jnp.int32)
    m0 = iot == 0

    def lat_expand(kr, xbuf):
        rfull = jnp.full((L,), kr, jnp.int32)
        rb = kr * PAD
        for s in range(NS - 1):
            xa = xbuf[kr, pl.ds(s * L, L)]
            xb = plsc.load_gather(xbuf, [rfull, iot1 + (s * L)])
            we = wlev[pl.ds(s * L, L)]
            wo = wlov[pl.ds(s * L, L)]
            d = xb - xa
            plsc.store_scatter(lbuf, [rb + 2 * s * L + iot2], xa + we * d)
            plsc.store_scatter(lbuf, [rb + 2 * s * L + 1 + iot2], xa + wo * d)
        s = NS - 1
        ia = jnp.minimum(iot + (s * L), NLAT_IN - 1)
        ib = jnp.minimum(iot1 + (s * L), NLAT_IN - 1)
        xa = plsc.load_gather(xbuf, [rfull, ia])
        xb = plsc.load_gather(xbuf, [rfull, ib])
        we = wlev[pl.ds(s * L, L)]
        wo = wlov[pl.ds(s * L, L)]
        d = xb - xa
        plsc.store_scatter(lbuf, [rb + 2 * s * L + iot2], xa + we * d)
        plsc.store_scatter(lbuf, [rb + 2 * s * L + 1 + iot2], xa + wo * d)

    NP = OROWS // 2

    def lon_tile(q0, ob):
        wes = [webv[pl.ds((q0 + pq) * L, L)] for pq in range(NP)]
        wos = [wobv[pl.ds((q0 + pq) * L, L)] for pq in range(NP)]
        qbs = [(q0 + pq) * PAD for pq in range(NP)]

        @plsc.parallel_loop(0, NT - 1)
        def _chunks(t):
            tb = t * L
            for pq in range(NP):
                a = lbuf[pl.ds(qbs[pq] + tb, L)]
                b = lbuf[pl.ds(qbs[pq] + PAD + tb, L)]
                d = b - a
                ob[2 * pq, pl.ds(tb, L)] = a + wes[pq] * d
                ob[2 * pq + 1, pl.ds(tb, L)] = a + wos[pq] * d

        for pq in range(NP):
            a = lbuf[pl.ds(qbs[pq] + (NT - 1) * L, L)]
            b = lbuf[pl.ds(qbs[pq] + PAD + (NT - 1) * L, L)]
            d = b - a
            plsc.store_scatter(ob, [jnp.full((L,), 2 * pq, jnp.int32), last],
                               a + wes[pq] * d, mask=m0)
            plsc.store_scatter(ob, [jnp.full((L,), 2 * pq + 1, jnp.int32), last],
                               a + wos[pq] * d, mask=m0)

    def stage(blk, ch, buf, sem):
        @pl.when(blk < NBLK - 1)
        def _main():
            pltpu.async_copy(x_hbm.at[ch, pl.ds(blk * KB, KB + 8), :], buf, sem)

        @pl.when(blk == NBLK - 1)
        def _wrap():
            pltpu.async_copy(x_hbm.at[ch, pl.ds(blk * KB, KB), :],
                             buf.at[pl.ds(0, KB), :], sem)
            pltpu.async_copy(x_hbm.at[ch, pl.ds(0, 8), :],
                             buf.at[pl.ds(KB, 8), :], sem)

    def per_channel(cc, _):
        ch = wid * CPW + cc
        stage(0, ch, xb0, semx0)

        def per_bpair(bp, _):
            for bb in range(2):
                blk = 2 * bp + bb
                xbuf = xb0 if bb == 0 else xb1
                semx = semx0 if bb == 0 else semx1
                nbuf = xb1 if bb == 0 else xb0
                nsem = semx1 if bb == 0 else semx0
                k0 = blk * KB
                pltpu.sync_copy(we_hbm.at[pl.ds(k0 * L, KB * L)], webv)
                pltpu.sync_copy(wo_hbm.at[pl.ds(k0 * L, KB * L)], wobv)
                pltpu.make_async_copy(
                    x_hbm.at[ch, pl.ds(0, KB + 8), :], xbuf, semx).wait()

                @pl.when(blk + 1 < NBLK)
                def _prefetch():
                    stage(blk + 1, ch, nbuf, nsem)

                @plsc.parallel_loop(0, KB + 1)
                def _do_lat(kr):
                    lat_expand(kr, xbuf)

                def ochunk(g2, _):
                    for gg in range(2):
                        g = 2 * g2 + gg
                        ob = ob0 if gg == 0 else ob1
                        sem = sem0 if gg == 0 else sem1
                        orow = 2 * k0 + OROWS * g

                        @pl.when(g2 > 0)
                        def _drain():
                            pltpu.make_async_copy(
                                ob,
                                out_hbm.at[ch, pl.ds(orow - 2 * OROWS, OROWS), :],
                                sem).wait()
                        lon_tile(NP * g, ob)
                        pltpu.async_copy(
                            ob, out_hbm.at[ch, pl.ds(orow, OROWS), :], sem)
                lax.fori_loop(0, NG // 2, ochunk, None)
                tail = 2 * k0 + (NG - 2) * OROWS
                pltpu.make_async_copy(
                    ob0, out_hbm.at[ch, pl.ds(tail, OROWS), :], sem0).wait()
                pltpu.make_async_copy(
                    ob1, out_hbm.at[ch, pl.ds(tail + OROWS, OROWS), :], sem1).wait()

        lax.fori_loop(0, NBLK // 2, per_bpair, None)

        for bb_last in [NBLK - 1]:
            blk = bb_last
            xbuf = xb0 if blk % 2 == 0 else xb1
            semx = semx0 if blk % 2 == 0 else semx1
            k0 = blk * KB
            pltpu.sync_copy(we_hbm.at[pl.ds(k0 * L, KB * L)], webv)
            pltpu.sync_copy(wo_hbm.at[pl.ds(k0 * L, KB * L)], wobv)
            pltpu.make_async_copy(
                x_hbm.at[ch, pl.ds(0, KB + 8), :], xbuf, semx).wait()

            @plsc.parallel_loop(0, KB + 1)
            def _do_lat(kr):
                lat_expand(kr, xbuf)

            def ochunk(g2, _):
                for gg in range(2):
                    g = 2 * g2 + gg
                    ob = ob0 if gg == 0 else ob1
                    sem = sem0 if gg == 0 else sem1
                    orow = 2 * k0 + OROWS * g

                    @pl.when(g2 > 0)
                    def _drain():
                        pltpu.make_async_copy(
                            ob,
                            out_hbm.at[ch, pl.ds(orow - 2 * OROWS, OROWS), :],
                            sem).wait()
                    lon_tile(NP * g, ob)
                    pltpu.async_copy(
                        ob, out_hbm.at[ch, pl.ds(orow, OROWS), :], sem)
            lax.fori_loop(0, NG // 2, ochunk, None)
            tail = 2 * k0 + (NG - 2) * OROWS
            pltpu.make_async_copy(
                ob0, out_hbm.at[ch, pl.ds(tail, OROWS), :], sem0).wait()
            pltpu.make_async_copy(
                ob1, out_hbm.at[ch, pl.ds(tail + OROWS, OROWS), :], sem1).wait()

    lax.fori_loop(0, CPW, per_channel, None)


@jax.jit
def _run(xT, wle, wlo, web, wob):
    mesh = plsc.VectorSubcoreMesh(core_axis_name="c", subcore_axis_name="s",
                                  num_cores=2, num_subcores=16)
    k = pl.kernel(
        _body,
        out_type=jax.ShapeDtypeStruct((C, NLON_OUT, NLAT_OUT), jnp.float32),
        mesh=mesh,
        compiler_params=pltpu.CompilerParams(
            needs_layout_passes=False, use_tc_tiling_on_sc=True),
        scratch_types=[
            pltpu.VMEM((KB + 8, NLAT_IN), jnp.float32),
            pltpu.VMEM((KB + 8, NLAT_IN), jnp.float32),
            pltpu.VMEM(((KB + 1) * PAD,), jnp.float32),
            pltpu.VMEM((OROWS, NLAT_OUT), jnp.float32),
            pltpu.VMEM((OROWS, NLAT_OUT), jnp.float32),
            pltpu.VMEM((SPAD,), jnp.float32),
            pltpu.VMEM((SPAD,), jnp.float32),
            pltpu.VMEM((KB * L,), jnp.float32),
            pltpu.VMEM((KB * L,), jnp.float32),
            pltpu.SemaphoreType.DMA,
            pltpu.SemaphoreType.DMA,
            pltpu.SemaphoreType.DMA,
            pltpu.SemaphoreType.DMA,
        ],
    )
    return k(xT, wle, wlo, web, wob)


def kernel(x, lat_idx, lat_weights, lon_idx_left, lon_idx_right, lon_weights):
    del lat_idx, lon_idx_left, lon_idx_right
    xT = jnp.transpose(x.reshape(C, NLAT_IN, NLON_IN), (0, 2, 1))
    wl = lat_weights.reshape(NLAT_OUT)
    wle = jnp.zeros((SPAD,), jnp.float32).at[: (NLAT_OUT + 1) // 2].set(wl[0::2])
    wlo = jnp.zeros((SPAD,), jnp.float32).at[: NLAT_OUT // 2].set(wl[1::2])
    web = jnp.broadcast_to(lon_weights[0::2][:, None], (NLON_IN, L)).reshape(-1)
    wob = jnp.broadcast_to(lon_weights[1::2][:, None], (NLON_IN, L)).reshape(-1)
    outT = _run(xT, wle, wlo, web, wob)
    return jnp.transpose(outT, (0, 2, 1)).reshape(1, C, NLAT_OUT, NLON_OUT)


# speedup vs baseline: 1.2333x; 1.2333x over previous
import jax
import jax.numpy as jnp
from jax import lax
from jax.experimental import pallas as pl
from jax.experimental.pallas import tpu as pltpu
from jax.experimental.pallas import tpu_sc as plsc

C = 64
NLAT_IN, NLON_IN = 361, 720
NLAT_OUT, NLON_OUT = 721, 1440
L = 16
NS = 23
SPAD = NS * L
PAD = 2 * SPAD
NT = PAD // L
KB = 40
NBLK = NLON_IN // KB
NW = 32
CPW = C // NW


def _body(x_hbm, wle_hbm, wlo_hbm, we_hbm, wo_hbm, out_hbm,
          xb0, xb1, lbuf, ob0, ob1, wlev, wlov, webv, wobv,
          sem0, sem1, semx0, semx1):
    cid = lax.axis_index("c")
    sid = lax.axis_index("s")
    wid = sid * 2 + cid

    pltpu.sync_copy(wle_hbm, wlev)
    pltpu.sync_copy(wlo_hbm, wlov)
    pltpu.sync_copy(we_hbm, webv)
    pltpu.sync_copy(wo_hbm, wobv)

    iot = lax.iota(jnp.int32, L)
    iot1 = iot + 1
    iot2 = 2 * iot
    last = jnp.full((L,), NLAT_OUT - 1, jnp.int32)
    m0 = iot == 0

    def lat_expand(kr, xbuf):
        rfull = jnp.full((L,), kr, jnp.int32)
        rb = kr * PAD
        for s in range(NS - 1):
            xa = xbuf[kr, pl.ds(s * L, L)]
            xb = plsc.load_gather(xbuf, [rfull, iot1 + (s * L)])
            we = wlev[pl.ds(s * L, L)]
            wo = wlov[pl.ds(s * L, L)]
            d = xb - xa
            plsc.store_scatter(lbuf, [rb + 2 * s * L + iot2], xa + we * d)
            plsc.store_scatter(lbuf, [rb + 2 * s * L + 1 + iot2], xa + wo * d)
        s = NS - 1
        ia = jnp.minimum(iot + (s * L), NLAT_IN - 1)
        ib = jnp.minimum(iot1 + (s * L), NLAT_IN - 1)
        xa = plsc.load_gather(xbuf, [rfull, ia])
        xb = plsc.load_gather(xbuf, [rfull, ib])
        we = wlev[pl.ds(s * L, L)]
        wo = wlov[pl.ds(s * L, L)]
        d = xb - xa
        plsc.store_scatter(lbuf, [rb + 2 * s * L + iot2], xa + we * d)
        plsc.store_scatter(lbuf, [rb + 2 * s * L + 1 + iot2], xa + wo * d)

    def lon_tile(qa0, q0, ob):
        wes = [webv[pl.ds((qa0 + pq) * L, L)] for pq in range(4)]
        wos = [wobv[pl.ds((qa0 + pq) * L, L)] for pq in range(4)]
        qbs = [(q0 + pq) * PAD for pq in range(4)]

        @plsc.parallel_loop(0, NT - 1)
        def _chunks(t):
            tb = t * L
            for pq in range(4):
                a = lbuf[pl.ds(qbs[pq] + tb, L)]
                b = lbuf[pl.ds(qbs[pq] + PAD + tb, L)]
                d = b - a
                ob[2 * pq, pl.ds(tb, L)] = a + wes[pq] * d
                ob[2 * pq + 1, pl.ds(tb, L)] = a + wos[pq] * d

        for pq in range(4):
            a = lbuf[pl.ds(qbs[pq] + (NT - 1) * L, L)]
            b = lbuf[pl.ds(qbs[pq] + PAD + (NT - 1) * L, L)]
            d = b - a
            plsc.store_scatter(ob, [jnp.full((L,), 2 * pq, jnp.int32), last],
                               a + wes[pq] * d, mask=m0)
            plsc.store_scatter(ob, [jnp.full((L,), 2 * pq + 1, jnp.int32), last],
                               a + wos[pq] * d, mask=m0)

    def stage(blk, ch, buf, sem):
        @pl.when(blk < NBLK - 1)
        def _main():
            pltpu.async_copy(x_hbm.at[ch, pl.ds(blk * KB, KB + 8), :], buf, sem)

        @pl.when(blk == NBLK - 1)
        def _wrap():
            pltpu.async_copy(x_hbm.at[ch, pl.ds(blk * KB, KB), :],
                             buf.at[pl.ds(0, KB), :], sem)
            pltpu.async_copy(x_hbm.at[ch, pl.ds(0, 8), :],
                             buf.at[pl.ds(KB, 8), :], sem)

    def per_channel(cc, _):
        ch = wid * CPW + cc
        stage(0, ch, xb0, semx0)

        def per_pair(bp, _):
            for bb in range(2):
                blk = 2 * bp + bb
                xbuf = xb0 if bb == 0 else xb1
                semx = semx0 if bb == 0 else semx1
                nbuf = xb1 if bb == 0 else xb0
                nsem = semx1 if bb == 0 else semx0
                k0 = blk * KB
                pltpu.make_async_copy(
                    x_hbm.at[ch, pl.ds(0, KB + 8), :], xbuf, semx).wait()

                @pl.when(blk + 1 < NBLK)
                def _prefetch():
                    stage(blk + 1, ch, nbuf, nsem)

                @plsc.parallel_loop(0, KB + 1)
                def _do_lat(kr):
                    lat_expand(kr, xbuf)

                def tilerow(g2, _):
                    for gg in range(2):
                        g = 2 * g2 + gg
                        ob = ob0 if gg == 0 else ob1
                        sem = sem0 if gg == 0 else sem1
                        orow = 2 * k0 + 8 * g

                        @pl.when(g2 > 0)
                        def _drain():
                            pltpu.make_async_copy(
                                ob, out_hbm.at[ch, pl.ds(orow - 16, 8), :], sem
                            ).wait()
                        lon_tile(k0 + 4 * g, 4 * g, ob)
                        pltpu.async_copy(
                            ob, out_hbm.at[ch, pl.ds(orow, 8), :], sem)
                lax.fori_loop(0, 5, tilerow, None)
                pltpu.make_async_copy(
                    ob0, out_hbm.at[ch, pl.ds(2 * k0 + 64, 8), :], sem0).wait()
                pltpu.make_async_copy(
                    ob1, out_hbm.at[ch, pl.ds(2 * k0 + 72, 8), :], sem1).wait()

        lax.fori_loop(0, NBLK // 2, per_pair, None)

    lax.fori_loop(0, CPW, per_channel, None)


@jax.jit
def _run(xT, wle, wlo, web, wob):
    mesh = plsc.VectorSubcoreMesh(core_axis_name="c", subcore_axis_name="s",
                                  num_cores=2, num_subcores=16)
    k = pl.kernel(
        _body,
        out_type=jax.ShapeDtypeStruct((C, NLON_OUT, NLAT_OUT), jnp.float32),
        mesh=mesh,
        compiler_params=pltpu.CompilerParams(
            needs_layout_passes=False, use_tc_tiling_on_sc=True),
        scratch_types=[
            pltpu.VMEM((KB + 8, NLAT_IN), jnp.float32),
            pltpu.VMEM((KB + 8, NLAT_IN), jnp.float32),
            pltpu.VMEM(((KB + 1) * PAD,), jnp.float32),
            pltpu.VMEM((8, NLAT_OUT), jnp.float32),
            pltpu.VMEM((8, NLAT_OUT), jnp.float32),
            pltpu.VMEM((SPAD,), jnp.float32),
            pltpu.VMEM((SPAD,), jnp.float32),
            pltpu.VMEM((NLON_IN * L,), jnp.float32),
            pltpu.VMEM((NLON_IN * L,), jnp.float32),
            pltpu.SemaphoreType.DMA,
            pltpu.SemaphoreType.DMA,
            pltpu.SemaphoreType.DMA,
            pltpu.SemaphoreType.DMA,
        ],
    )
    return k(xT, wle, wlo, web, wob)


def kernel(x, lat_idx, lat_weights, lon_idx_left, lon_idx_right, lon_weights):
    del lat_idx, lon_idx_left, lon_idx_right
    xT = jnp.transpose(x.reshape(C, NLAT_IN, NLON_IN), (0, 2, 1))
    wl = lat_weights.reshape(NLAT_OUT)
    wle = jnp.zeros((SPAD,), jnp.float32).at[: (NLAT_OUT + 1) // 2].set(wl[0::2])
    wlo = jnp.zeros((SPAD,), jnp.float32).at[: NLAT_OUT // 2].set(wl[1::2])
    web = jnp.broadcast_to(lon_weights[0::2][:, None], (NLON_IN, L)).reshape(-1)
    wob = jnp.broadcast_to(lon_weights[1::2][:, None], (NLON_IN, L)).reshape(-1)
    outT = _run(xT, wle, wlo, web, wob)
    return jnp.transpose(outT, (0, 2, 1)).reshape(1, C, NLAT_OUT, NLON_OUT)
